# baseline (speedup 1.0000x reference)
import functools

import numpy as np
import jax
import jax.numpy as jnp
from jax import lax
from jax.experimental import pallas as pl
from jax.experimental.pallas import tpu as pltpu
from jax.experimental.pallas import tpu_sc as plsc

_B = 4096
_NNEG = 128
_P = 4
_D = 128
_R = 512
_NW = 32
_RPW = _B // _NW


@functools.lru_cache(maxsize=1)
def _neg_idx():
    rng = np.random.default_rng(0)
    all_idx = np.arange(_B)
    neg = np.stack([rng.choice(np.delete(all_idx, i), _NNEG, replace=False)
                    for i in range(_B)])
    return neg


@functools.lru_cache(maxsize=1)
def _sc_idx():
    neg = _neg_idx()
    flat = (np.arange(_B)[:, None] * _B + neg).astype(np.int32)
    idx3 = np.transpose(flat.reshape(_NW, _RPW, _NNEG), (0, 2, 1)).copy()
    return jnp.asarray(idx3)


def _k1_body(z_ref, pz_ref, s_ref, pe_ref, zn_ref, znb_ref):
    i = pl.program_id(0)

    @pl.when(i == 0)
    def _init():
        z = z_ref[...]
        n2 = jnp.sum(z * z, axis=1, keepdims=True)
        zn = z * lax.rsqrt(jnp.maximum(n2, 1e-24))
        zn_ref[...] = zn
        znb_ref[...] = zn.astype(jnp.bfloat16)

    zn_blk = zn_ref[pl.ds(i * _R, _R), :]
    s_ref[...] = jax.lax.dot_general(
        zn_blk.astype(jnp.bfloat16), znb_ref[...],
        (((1,), (1,)), ((), ())), preferred_element_type=jnp.float32)

    p2 = pz_ref[...]
    zrep = jnp.concatenate([zn_blk] * _P, axis=1)
    idx = lax.broadcasted_iota(jnp.int32, (_P * _D, _P), 0) // _D
    w = (idx == lax.broadcasted_iota(jnp.int32, (_P * _D, _P), 1)).astype(jnp.float32)
    pn2 = jax.lax.dot_general(p2 * p2, w, (((1,), (0,)), ((), ())),
                              preferred_element_type=jnp.float32)
    pd = jax.lax.dot_general(p2 * zrep, w, (((1,), (0,)), ((), ())),
                             preferred_element_type=jnp.float32)
    pos_s = pd * lax.rsqrt(jnp.maximum(pn2, 1e-24))
    pe_ref[...] = jnp.sum(jnp.exp(pos_s), axis=1, keepdims=True)


def _k1(z, pz2):
    return pl.pallas_call(
        _k1_body,
        grid=(_B // _R,),
        in_specs=[
            pl.BlockSpec((_B, _D), lambda i: (0, 0)),
            pl.BlockSpec((_R, _P * _D), lambda i: (i, 0)),
        ],
        out_specs=[
            pl.BlockSpec((_R, _B), lambda i: (i, 0)),
            pl.BlockSpec((_R, 1), lambda i: (i, 0)),
        ],
        out_shape=[
            jax.ShapeDtypeStruct((_B, _B), jnp.float32),
            jax.ShapeDtypeStruct((_B, 1), jnp.float32),
        ],
        scratch_shapes=[
            pltpu.VMEM((_B, _D), jnp.float32),
            pltpu.VMEM((_B, _D), jnp.bfloat16),
        ],
    )(z, pz2)


_CHUNK = 16


def _k2_body(s_hbm, idx_hbm, out_hbm, idx_v, val_v, acc_v, sem):
    w = lax.axis_index("s") * 2 + lax.axis_index("c")
    pltpu.sync_copy(idx_hbm.at[w], idx_v)

    def chunk(c, _):
        base = c * _CHUNK
        for k in range(_CHUNK):
            pltpu.async_copy(s_hbm.at[idx_v.at[base + k]],
                             val_v.at[k], sem)
        for k in range(_CHUNK):
            pltpu.make_async_copy(s_hbm.at[idx_v.at[base + k]],
                                  val_v.at[k], sem).wait()
        for k in range(_CHUNK):
            for j in range(_RPW // 16):
                sl = pl.ds(j * 16, 16)
                acc_v[sl] = acc_v[sl] + jnp.exp(val_v[k, sl])
        return _

    for j in range(_RPW // 16):
        acc_v[pl.ds(j * 16, 16)] = jnp.zeros((16,), jnp.float32)
    lax.fori_loop(0, _NNEG // _CHUNK, chunk, 0)
    pltpu.sync_copy(acc_v, out_hbm.at[pl.ds(w * _RPW, _RPW)])


def _k2(s_flat):
    mesh = plsc.VectorSubcoreMesh(core_axis_name="c", subcore_axis_name="s")
    kfn = functools.partial(
        pl.kernel, mesh=mesh,
        out_type=jax.ShapeDtypeStruct((_B,), jnp.float32),
        scratch_types=[
            pltpu.VMEM((_NNEG, _RPW), jnp.int32),
            pltpu.VMEM((_CHUNK, _RPW), jnp.float32),
            pltpu.VMEM((_RPW,), jnp.float32),
            pltpu.SemaphoreType.DMA,
        ],
    )(_k2_body)
    return kfn(s_flat, _sc_idx())


def _k3_body(ne_ref, pe_ref, out_ref):
    ne = ne_ref[...]
    pe = pe_ref[...]
    loss = jnp.log(ne + pe) - jnp.log(pe)
    out_ref[...] = jnp.full((1, 1), jnp.sum(loss) * (1.0 / _B), jnp.float32)


def _k3(neg_e, pos_e):
    return pl.pallas_call(
        _k3_body,
        in_specs=[pl.BlockSpec((_NW, _RPW), lambda: (0, 0)),
                  pl.BlockSpec((_NW, _RPW), lambda: (0, 0))],
        out_specs=pl.BlockSpec((1, 1), lambda: (0, 0)),
        out_shape=jax.ShapeDtypeStruct((1, 1), jnp.float32),
    )(neg_e, pos_e)


def kernel(z_vecs, pos_z_vecs):
    pz2 = jnp.reshape(pos_z_vecs, (_B, _P * _D))
    s, pos_e = _k1(z_vecs, pz2)
    neg_e = _k2(jnp.reshape(s, (_B * _B,)))
    out = _k3(jnp.reshape(neg_e, (_NW, _RPW)),
              jnp.reshape(pos_e, (_NW, _RPW)))
    return jnp.reshape(out, ())


# speedup vs baseline: 1.0613x; 1.0613x over previous
import functools

import numpy as np
import jax
import jax.numpy as jnp
from jax import lax
from jax.experimental import pallas as pl
from jax.experimental.pallas import tpu as pltpu
from jax.experimental.pallas import tpu_sc as plsc

_B = 4096
_NNEG = 128
_P = 4
_D = 128
_R = 512
_NW = 32
_RPW = _B // _NW


@functools.lru_cache(maxsize=1)
def _neg_idx():
    rng = np.random.default_rng(0)
    all_idx = np.arange(_B)
    neg = np.stack([rng.choice(np.delete(all_idx, i), _NNEG, replace=False)
                    for i in range(_B)])
    return neg


@functools.lru_cache(maxsize=1)
def _sc_idx():
    neg = _neg_idx()
    flat = (np.arange(_B)[:, None] * _B + neg).astype(np.int32)
    idx3 = np.transpose(flat.reshape(_NW, _RPW, _NNEG), (0, 2, 1)).copy()
    return jnp.asarray(idx3)


def _k1_body(z_ref, pz_ref, s_ref, pe_ref, zn_ref, znb_ref):
    i = pl.program_id(0)

    @pl.when(i == 0)
    def _init():
        z = z_ref[...]
        n2 = jnp.sum(z * z, axis=1, keepdims=True)
        zn = z * lax.rsqrt(jnp.maximum(n2, 1e-24))
        zn_ref[...] = zn
        znb_ref[...] = zn.astype(jnp.bfloat16)

    zn_blk = zn_ref[pl.ds(i * _R, _R), :]
    s_ref[...] = jax.lax.dot_general(
        zn_blk.astype(jnp.bfloat16), znb_ref[...],
        (((1,), (1,)), ((), ())), preferred_element_type=jnp.float32)

    p3 = pz_ref[...]
    pn2 = jnp.sum(p3 * p3, axis=2)
    pd = jnp.sum(p3 * zn_blk[:, None, :], axis=2)
    pos_s = pd * lax.rsqrt(jnp.maximum(pn2, 1e-24))
    pe_ref[...] = jnp.sum(jnp.exp(pos_s), axis=1, keepdims=True)


def _k1(z, pz):
    return pl.pallas_call(
        _k1_body,
        grid=(_B // _R,),
        in_specs=[
            pl.BlockSpec((_B, _D), lambda i: (0, 0)),
            pl.BlockSpec((_R, _P, _D), lambda i: (i, 0, 0)),
        ],
        out_specs=[
            pl.BlockSpec((_R, _B), lambda i: (i, 0)),
            pl.BlockSpec((_R, 1), lambda i: (i, 0)),
        ],
        out_shape=[
            jax.ShapeDtypeStruct((_B, _B), jnp.float32),
            jax.ShapeDtypeStruct((_B, 1), jnp.float32),
        ],
        scratch_shapes=[
            pltpu.VMEM((_B, _D), jnp.float32),
            pltpu.VMEM((_B, _D), jnp.bfloat16),
        ],
    )(z, pz)


_CHUNK = 16


def _k2_body(s_hbm, idx_hbm, out_hbm, idx_v, val_v, acc_v, sem):
    w = lax.axis_index("s") * 2 + lax.axis_index("c")
    pltpu.sync_copy(idx_hbm.at[w], idx_v)

    def chunk(c, _):
        base = c * _CHUNK
        for k in range(_CHUNK):
            pltpu.async_copy(s_hbm.at[idx_v.at[base + k]],
                             val_v.at[k], sem)
        for k in range(_CHUNK):
            pltpu.make_async_copy(s_hbm.at[idx_v.at[base + k]],
                                  val_v.at[k], sem).wait()
        for k in range(_CHUNK):
            for j in range(_RPW // 16):
                sl = pl.ds(j * 16, 16)
                acc_v[sl] = acc_v[sl] + jnp.exp(val_v[k, sl])
        return _

    for j in range(_RPW // 16):
        acc_v[pl.ds(j * 16, 16)] = jnp.zeros((16,), jnp.float32)
    lax.fori_loop(0, _NNEG // _CHUNK, chunk, 0)
    pltpu.sync_copy(acc_v, out_hbm.at[pl.ds(w * _RPW, _RPW)])


def _k2(s_2d):
    mesh = plsc.VectorSubcoreMesh(core_axis_name="c", subcore_axis_name="s")
    kfn = functools.partial(
        pl.kernel, mesh=mesh,
        out_type=jax.ShapeDtypeStruct((_B,), jnp.float32),
        scratch_types=[
            pltpu.VMEM((_NNEG, _RPW), jnp.int32),
            pltpu.VMEM((_CHUNK, _RPW), jnp.float32),
            pltpu.VMEM((_RPW,), jnp.float32),
            pltpu.SemaphoreType.DMA,
        ],
    )(_k2_body)
    return kfn(s_2d, _sc_idx())


def _k3_body(ne_ref, pe_ref, out_ref):
    ne = ne_ref[...]
    pe = pe_ref[...][:, 0]
    loss = jnp.log(ne + pe) - jnp.log(pe)
    out_ref[...] = jnp.full((1, 1), jnp.sum(loss) * (1.0 / _B), jnp.float32)


def _k3(neg_e, pos_e):
    return pl.pallas_call(
        _k3_body,
        in_specs=[pl.BlockSpec((_B,), lambda: (0,)),
                  pl.BlockSpec((_B, 1), lambda: (0, 0))],
        out_specs=pl.BlockSpec((1, 1), lambda: (0, 0)),
        out_shape=jax.ShapeDtypeStruct((1, 1), jnp.float32),
    )(neg_e, pos_e)


def kernel(z_vecs, pos_z_vecs):
    s, pos_e = _k1(z_vecs, pos_z_vecs)
    neg_e = _k2(jnp.reshape(s, (_B * _B,)))
    out = _k3(neg_e, pos_e)
    return jnp.reshape(out, ())


# speedup vs baseline: 1.6143x; 1.5210x over previous
import functools

import numpy as np
import jax
import jax.numpy as jnp
from jax import lax
from jax.experimental import pallas as pl
from jax.experimental.pallas import tpu as pltpu
from jax.experimental.pallas import tpu_sc as plsc

_B = 4096
_NNEG = 128
_P = 4
_D = 128
_R = 512
_NW = 32
_RPW = _B // _NW


@functools.lru_cache(maxsize=1)
def _neg_idx():
    rng = np.random.default_rng(0)
    all_idx = np.arange(_B)
    neg = np.stack([rng.choice(np.delete(all_idx, i), _NNEG, replace=False)
                    for i in range(_B)])
    return neg


@functools.lru_cache(maxsize=1)
def _sc_idx():
    neg = _neg_idx()
    flat = (np.arange(_B)[:, None] * _B + neg).astype(np.int32)
    idx3 = np.transpose(flat.reshape(_NW, _RPW, _NNEG), (0, 2, 1)).copy()
    return jnp.asarray(idx3)


def _k1_body(z_ref, pz_ref, s_ref, pe_ref, zn_ref, znb_ref):
    i = pl.program_id(0)

    @pl.when(i == 0)
    def _init():
        z = z_ref[...]
        n2 = jnp.sum(z * z, axis=1, keepdims=True)
        zn = z * lax.rsqrt(jnp.maximum(n2, 1e-24))
        zn_ref[...] = zn
        znb_ref[...] = zn.astype(jnp.bfloat16)

    zn_blk = zn_ref[pl.ds(i * _R, _R), :]
    s2 = jax.lax.dot_general(
        zn_blk.astype(jnp.bfloat16), znb_ref[...],
        (((1,), (1,)), ((), ())), preferred_element_type=jnp.float32)
    s_ref[...] = jnp.reshape(s2, (_R * _B,))

    p3 = pz_ref[...]
    pn2 = jnp.sum(p3 * p3, axis=2)
    pd = jnp.sum(p3 * zn_blk[:, None, :], axis=2)
    pos_s = pd * lax.rsqrt(jnp.maximum(pn2, 1e-24))
    pe_ref[...] = jnp.sum(jnp.exp(pos_s), axis=1, keepdims=True)


def _k1(z, pz):
    return pl.pallas_call(
        _k1_body,
        grid=(_B // _R,),
        in_specs=[
            pl.BlockSpec((_B, _D), lambda i: (0, 0)),
            pl.BlockSpec((_R, _P, _D), lambda i: (i, 0, 0)),
        ],
        out_specs=[
            pl.BlockSpec((_R * _B,), lambda i: (i,)),
            pl.BlockSpec((_R, 1), lambda i: (i, 0)),
        ],
        out_shape=[
            jax.ShapeDtypeStruct((_B * _B,), jnp.float32),
            jax.ShapeDtypeStruct((_B, 1), jnp.float32),
        ],
        scratch_shapes=[
            pltpu.VMEM((_B, _D), jnp.float32),
            pltpu.VMEM((_B, _D), jnp.bfloat16),
        ],
    )(z, pz)


_CHUNK = 16


def _k2_body(s_hbm, idx_hbm, out_hbm, idx_v, val_v, acc_v, sem):
    w = lax.axis_index("s") * 2 + lax.axis_index("c")
    pltpu.sync_copy(idx_hbm.at[w], idx_v)

    def chunk(c, _):
        base = c * _CHUNK
        for k in range(_CHUNK):
            pltpu.async_copy(s_hbm.at[idx_v.at[base + k]],
                             val_v.at[k], sem)
        for k in range(_CHUNK):
            pltpu.make_async_copy(s_hbm.at[idx_v.at[base + k]],
                                  val_v.at[k], sem).wait()
        for k in range(_CHUNK):
            for j in range(_RPW // 16):
                sl = pl.ds(j * 16, 16)
                acc_v[sl] = acc_v[sl] + jnp.exp(val_v[k, sl])
        return _

    for j in range(_RPW // 16):
        acc_v[pl.ds(j * 16, 16)] = jnp.zeros((16,), jnp.float32)
    lax.fori_loop(0, _NNEG // _CHUNK, chunk, 0)
    pltpu.sync_copy(acc_v, out_hbm.at[pl.ds(w * _RPW, _RPW)])


def _k2(s_2d):
    mesh = plsc.VectorSubcoreMesh(core_axis_name="c", subcore_axis_name="s")
    kfn = functools.partial(
        pl.kernel, mesh=mesh,
        out_type=jax.ShapeDtypeStruct((_B,), jnp.float32),
        scratch_types=[
            pltpu.VMEM((_NNEG, _RPW), jnp.int32),
            pltpu.VMEM((_CHUNK, _RPW), jnp.float32),
            pltpu.VMEM((_RPW,), jnp.float32),
            pltpu.SemaphoreType.DMA,
        ],
    )(_k2_body)
    return kfn(s_2d, _sc_idx())


def _k3_body(ne_ref, pe_ref, out_ref):
    ne = ne_ref[...]
    pe = pe_ref[...][:, 0]
    loss = jnp.log(ne + pe) - jnp.log(pe)
    out_ref[...] = jnp.full((1, 1), jnp.sum(loss) * (1.0 / _B), jnp.float32)


def _k3(neg_e, pos_e):
    return pl.pallas_call(
        _k3_body,
        in_specs=[pl.BlockSpec((_B,), lambda: (0,)),
                  pl.BlockSpec((_B, 1), lambda: (0, 0))],
        out_specs=pl.BlockSpec((1, 1), lambda: (0, 0)),
        out_shape=jax.ShapeDtypeStruct((1, 1), jnp.float32),
    )(neg_e, pos_e)


def kernel(z_vecs, pos_z_vecs):
    s, pos_e = _k1(z_vecs, pos_z_vecs)
    neg_e = _k2(s)
    out = _k3(neg_e, pos_e)
    return jnp.reshape(out, ())


# speedup vs baseline: 1.6219x; 1.0047x over previous
import functools

import numpy as np
import jax
import jax.numpy as jnp
from jax import lax
from jax.experimental import pallas as pl
from jax.experimental.pallas import tpu as pltpu
from jax.experimental.pallas import tpu_sc as plsc

_B = 4096
_NNEG = 128
_P = 4
_D = 128
_R = 512
_NW = 32
_RPW = _B // _NW


@functools.lru_cache(maxsize=1)
def _neg_idx():
    rng = np.random.default_rng(0)
    all_idx = np.arange(_B)
    neg = np.stack([rng.choice(np.delete(all_idx, i), _NNEG, replace=False)
                    for i in range(_B)])
    return neg


@functools.lru_cache(maxsize=1)
def _sc_idx():
    neg = _neg_idx()
    flat = (np.arange(_B)[:, None] * _B + neg).astype(np.int32)
    idx3 = np.transpose(flat.reshape(_NW, _RPW, _NNEG), (0, 2, 1)).copy()
    return jnp.asarray(idx3)


def _k1_body(z_ref, pz_ref, s_ref, pe_ref, zn_ref, znb_ref):
    i = pl.program_id(0)

    @pl.when(i == 0)
    def _init():
        z = z_ref[...]
        n2 = jnp.sum(z * z, axis=1, keepdims=True)
        zn = z * lax.rsqrt(jnp.maximum(n2, 1e-24))
        zn_ref[...] = zn
        znb_ref[...] = zn.astype(jnp.bfloat16)

    zn_blk = zn_ref[pl.ds(i * _R, _R), :]
    s2 = jax.lax.dot_general(
        zn_blk.astype(jnp.bfloat16), znb_ref[...],
        (((1,), (1,)), ((), ())), preferred_element_type=jnp.float32)
    s_ref[...] = jnp.reshape(s2, (_R * _B,))

    p3 = pz_ref[...]
    pn2 = jnp.sum(p3 * p3, axis=2)
    pd = jnp.sum(p3 * zn_blk[:, None, :], axis=2)
    pos_s = pd * lax.rsqrt(jnp.maximum(pn2, 1e-24))
    pe_ref[...] = jnp.sum(jnp.exp(pos_s), axis=1, keepdims=True)


def _k1(z, pz):
    return pl.pallas_call(
        _k1_body,
        grid=(_B // _R,),
        in_specs=[
            pl.BlockSpec((_B, _D), lambda i: (0, 0)),
            pl.BlockSpec((_R, _P, _D), lambda i: (i, 0, 0)),
        ],
        out_specs=[
            pl.BlockSpec((_R * _B,), lambda i: (i,)),
            pl.BlockSpec((_R, 1), lambda i: (i, 0)),
        ],
        out_shape=[
            jax.ShapeDtypeStruct((_B * _B,), jnp.float32),
            jax.ShapeDtypeStruct((_B, 1), jnp.float32),
        ],
        scratch_shapes=[
            pltpu.VMEM((_B, _D), jnp.float32),
            pltpu.VMEM((_B, _D), jnp.bfloat16),
        ],
    )(z, pz)


_CHUNK = 16


def _k2_body(s_hbm, idx_hbm, out_hbm, idx_v, val_v, acc_v, sem):
    w = lax.axis_index("s") * 2 + lax.axis_index("c")
    pltpu.sync_copy(idx_hbm.at[w], idx_v)

    def chunk(c, _):
        base = c * _CHUNK
        for k in range(_CHUNK):
            pltpu.async_copy(s_hbm.at[idx_v.at[base + k]],
                             val_v.at[k], sem)
        for k in range(_CHUNK):
            pltpu.make_async_copy(s_hbm.at[idx_v.at[base + k]],
                                  val_v.at[k], sem).wait()
        for k in range(_CHUNK):
            for j in range(_RPW // 16):
                sl = pl.ds(j * 16, 16)
                acc_v[sl] = acc_v[sl] + jnp.exp(val_v[k, sl])
        return _

    for j in range(_RPW // 16):
        acc_v[pl.ds(j * 16, 16)] = jnp.zeros((16,), jnp.float32)
    lax.fori_loop(0, _NNEG // _CHUNK, chunk, 0)
    pltpu.sync_copy(acc_v, out_hbm.at[pl.ds(w * _RPW, _RPW)])


def _k2(s_flat):
    mesh = plsc.VectorSubcoreMesh(core_axis_name="c", subcore_axis_name="s")
    kfn = functools.partial(
        pl.kernel, mesh=mesh,
        out_type=jax.ShapeDtypeStruct((_B,), jnp.float32),
        scratch_types=[
            pltpu.VMEM((_NNEG, _RPW), jnp.int32),
            pltpu.VMEM((_CHUNK, _RPW), jnp.float32),
            pltpu.VMEM((_RPW,), jnp.float32),
            pltpu.SemaphoreType.DMA,
        ],
    )(_k2_body)
    return kfn(s_flat, _sc_idx())


def _k3_body(ne_ref, pe_ref, out_ref):
    ne = ne_ref[...]
    pe = pe_ref[...][:, 0]
    loss = jnp.log(ne + pe) - jnp.log(pe)
    out_ref[...] = jnp.full((1, 1), jnp.sum(loss) * (1.0 / _B), jnp.float32)


def _k3(neg_e, pos_e):
    return pl.pallas_call(
        _k3_body,
        in_specs=[pl.BlockSpec((_B,), lambda: (0,)),
                  pl.BlockSpec((_B, 1), lambda: (0, 0))],
        out_specs=pl.BlockSpec((1, 1), lambda: (0, 0)),
        out_shape=jax.ShapeDtypeStruct((1, 1), jnp.float32),
    )(neg_e, pos_e)


def kernel(z_vecs, pos_z_vecs):
    s, pos_e = _k1(z_vecs, pos_z_vecs)
    neg_e = _k2(s)
    out = _k3(neg_e, pos_e)
    return jnp.reshape(out, ())
